# Initial kernel scaffold; baseline (speedup 1.0000x reference)
import jax
import jax.numpy as jnp
from jax.experimental import pallas as pl


def kernel(x_cat, x_cont, tables, W1, b1, W2, b2, W3, b3):
    raise NotImplementedError("write your pallas kernel here")


# speedup vs baseline: 8.0750x; 8.0750x over previous
import functools

import jax
import jax.numpy as jnp
from jax import lax
from jax.experimental import pallas as pl
from jax.experimental.pallas import tpu as pltpu
from jax.experimental.pallas import tpu_sc as plsc

_N_FIELDS = 26
_VOCAB = 100000
_EMB = 32
_N_CONT = 13
_B = 16384
_H1 = 256
_H2 = 128

_NW = 32
_R = _B * _N_FIELDS
_RPW = _R // _NW
_NCH = 8
_CH = _RPW // _NCH

_BM = 2048


def _gather_body(table_hbm, idx_hbm, out_hbm, idx_v, rows_v, sem):
    wid = lax.axis_index("s") * 2 + lax.axis_index("c")
    base = wid * _RPW
    pltpu.sync_copy(idx_hbm.at[wid], idx_v)

    def step(c, carry):
        cb = c * _CH
        pltpu.async_copy(table_hbm.at[idx_v.at[c]], rows_v, sem).wait()
        pltpu.sync_copy(rows_v, out_hbm.at[pl.ds(base + cb, _CH)])
        return carry

    lax.fori_loop(0, _NCH, step, 0)


_gather = functools.partial(
    pl.kernel,
    out_type=jax.ShapeDtypeStruct((_R, _EMB), jnp.float32),
    mesh=plsc.VectorSubcoreMesh(core_axis_name="c", subcore_axis_name="s"),
    compiler_params=pltpu.CompilerParams(use_tc_tiling_on_sc=False),
    scratch_types=[
        pltpu.VMEM((_NCH, _CH), jnp.int32),
        pltpu.VMEM((_CH, _EMB), jnp.float32),
        pltpu.SemaphoreType.DMA,
    ],
)(_gather_body)


def _mlp_body(e_ref, xc_ref, w1a_ref, w1b_ref, b1_ref, w2_ref, b2_ref,
              w3_ref, b3_ref, o_ref):
    x1 = jnp.dot(e_ref[...], w1a_ref[...], preferred_element_type=jnp.float32)
    x1 = x1 + jnp.dot(xc_ref[...], w1b_ref[...],
                      preferred_element_type=jnp.float32)
    h1 = jax.nn.sigmoid(x1 + b1_ref[...])
    h2 = jax.nn.sigmoid(
        jnp.dot(h1, w2_ref[...], preferred_element_type=jnp.float32)
        + b2_ref[...])
    o_ref[...] = jax.nn.sigmoid(
        jnp.dot(h2, w3_ref[...], preferred_element_type=jnp.float32)
        + b3_ref[...])


def _mlp(e, xc, w1a, w1b, b1, w2, b2, w3, b3):
    n_in_e = _N_FIELDS * _EMB
    grid = _B // _BM
    return pl.pallas_call(
        _mlp_body,
        grid=(grid,),
        in_specs=[
            pl.BlockSpec((_BM, n_in_e), lambda i: (i, 0)),
            pl.BlockSpec((_BM, _N_CONT), lambda i: (i, 0)),
            pl.BlockSpec((n_in_e, _H1), lambda i: (0, 0)),
            pl.BlockSpec((_N_CONT, _H1), lambda i: (0, 0)),
            pl.BlockSpec((1, _H1), lambda i: (0, 0)),
            pl.BlockSpec((_H1, _H2), lambda i: (0, 0)),
            pl.BlockSpec((1, _H2), lambda i: (0, 0)),
            pl.BlockSpec((_H2, 1), lambda i: (0, 0)),
            pl.BlockSpec((1, 1), lambda i: (0, 0)),
        ],
        out_specs=pl.BlockSpec((_BM, 1), lambda i: (i, 0)),
        out_shape=jax.ShapeDtypeStruct((_B, 1), jnp.float32),
    )(e, xc, w1a, w1b, b1, w2, b2, w3, b3)


def kernel(x_cat, x_cont, tables, W1, b1, W2, b2, W3, b3):
    offs = (jnp.arange(_N_FIELDS, dtype=jnp.int32) * _VOCAB)[None, :]
    idx = (x_cat.astype(jnp.int32) + offs).reshape(_NW, _NCH, _CH)
    table_flat = tables.reshape(_N_FIELDS * _VOCAB, _EMB)

    embs = _gather(table_flat, idx)
    e = embs.reshape(_B, _N_FIELDS * _EMB)

    w1a = W1[: _N_FIELDS * _EMB]
    w1b = W1[_N_FIELDS * _EMB:]
    out = _mlp(e, x_cont, w1a, w1b, b1[None, :], W2, b2[None, :], W3,
               b3[None, :])
    return out
